# baseline (speedup 1.0000x reference)
import functools

import jax
import jax.numpy as jnp
from jax import lax
from jax.experimental import pallas as pl
from jax.experimental.pallas import tpu as pltpu
from jax.experimental.pallas import tpu_sc as plsc

N = 10000
F = 128
H = 128
G = 64
EPS = 1e-5

NC = 2
NS = 16
CH = 80
E_TOTAL = 320000
ITERS = E_TOTAL // (NC * NS) // CH

STG = 80
NSTG = N // STG
STG_ROUNDS = (NSTG + NS - 1) // NS

_MESH = plsc.VectorSubcoreMesh(
    core_axis_name="c", subcore_axis_name="s", num_cores=NC, num_subcores=NS
)


def _deg_body(dst3_hbm, e0_hbm, z16_hbm, degp_hbm, didx_all, ones_v, stage_v,
              dwide):
    c = lax.axis_index("c")
    s = lax.axis_index("s")
    iters = dst3_hbm.shape[1]
    wid = c * NS + s

    pltpu.sync_copy(e0_hbm, ones_v)
    pltpu.sync_copy(z16_hbm, stage_v)
    pltpu.sync_copy(dst3_hbm.at[wid], didx_all)

    if True:
        for ch in range(STG_ROUNDS):
            k = s + NS * ch

            @pl.when(k < NSTG)
            def _():
                pltpu.sync_copy(stage_v, dwide.at[pl.ds(k * STG, STG)])

        plsc.subcore_barrier()

        def body(i, carry):
            pltpu.sync_copy(ones_v, dwide.at[didx_all.at[i]], add=True)
            return carry

        lax.fori_loop(0, iters, body, 0)
        plsc.subcore_barrier()

        for ch in range(STG_ROUNDS):
            k = s + NS * ch

            @pl.when(k < NSTG)
            def _():
                pltpu.sync_copy(dwide.at[pl.ds(k * STG, STG)], stage_v)
                pltpu.sync_copy(stage_v, degp_hbm.at[c, pl.ds(k * STG, STG)])


_deg_call = pl.kernel(
    _deg_body,
    out_type=jax.ShapeDtypeStruct((NC, N, 16), jnp.float32),
    mesh=_MESH,
    scratch_types=[
        pltpu.VMEM((ITERS, CH), jnp.int32),
        pltpu.VMEM((CH, 16), jnp.float32),
        pltpu.VMEM((STG, 16), jnp.float32),
        pltpu.VMEM_SHARED((N, 16), jnp.float32),
    ],
    compiler_params=pltpu.CompilerParams(use_tc_tiling_on_sc=False),
)


def _scatter_body(hp_hbm, src3_hbm, dst3_hbm, z128_hbm, part_hbm,
                  sidx_all, didx_all, rows0, rows1, rows2, acc,
                  semg0, semg1, semg2):
    c = lax.axis_index("c")
    s = lax.axis_index("s")
    iters = src3_hbm.shape[1]
    assert iters % 3 == 2, "edge pipeline expects 3k+2 chunks"
    trips = iters // 3
    wid = c * NS + s

    pltpu.sync_copy(z128_hbm, rows0)
    pltpu.sync_copy(src3_hbm.at[wid], sidx_all)
    pltpu.sync_copy(dst3_hbm.at[wid], didx_all)

    if True:
        for ch in range(STG_ROUNDS):
            k = s + NS * ch

            @pl.when((k < NSTG) & (c == 0))
            def _():
                pltpu.sync_copy(hp_hbm.at[pl.ds(k * STG, STG)], rows0)
                pltpu.sync_copy(rows0, acc.at[pl.ds(k * STG, STG)])

            @pl.when((k < NSTG) & (c != 0))
            def _():
                pltpu.sync_copy(rows0, acc.at[pl.ds(k * STG, STG)])

        plsc.subcore_barrier()

        pltpu.async_copy(hp_hbm.at[sidx_all.at[0]], rows0, semg0)
        pltpu.async_copy(hp_hbm.at[sidx_all.at[1]], rows1, semg1)
        pltpu.async_copy(hp_hbm.at[sidx_all.at[2]], rows2, semg2)

        def step(i, rw, sem, refill):
            pltpu.make_async_copy(hp_hbm.at[sidx_all.at[i]], rw, sem).wait()
            pltpu.sync_copy(rw, acc.at[didx_all.at[i]], add=True)
            if refill is None:
                pltpu.async_copy(hp_hbm.at[sidx_all.at[i + 3]], rw, sem)
            else:
                @pl.when(refill)
                def _():
                    pltpu.async_copy(hp_hbm.at[sidx_all.at[i + 3]], rw, sem)

        def body(j, carry):
            i0 = 3 * j
            step(i0, rows0, semg0, None)
            step(i0 + 1, rows1, semg1, None)
            step(i0 + 2, rows2, semg2, j < trips - 1)
            return carry

        lax.fori_loop(0, trips, body, 0)
        pltpu.make_async_copy(hp_hbm.at[sidx_all.at[iters - 2]], rows0, semg0).wait()
        pltpu.sync_copy(rows0, acc.at[didx_all.at[iters - 2]], add=True)
        pltpu.make_async_copy(hp_hbm.at[sidx_all.at[iters - 1]], rows1, semg1).wait()
        pltpu.sync_copy(rows1, acc.at[didx_all.at[iters - 1]], add=True)
        plsc.subcore_barrier()

        for ch in range(STG_ROUNDS):
            k = s + NS * ch

            @pl.when(k < NSTG)
            def _():
                pltpu.sync_copy(acc.at[pl.ds(k * STG, STG)], rows0)
                pltpu.sync_copy(rows0, part_hbm.at[c, pl.ds(k * STG, STG)])


_scatter_call = pl.kernel(
    _scatter_body,
    out_type=jax.ShapeDtypeStruct((NC, N, H), jnp.float32),
    mesh=_MESH,
    scratch_types=[
        pltpu.VMEM((ITERS, CH), jnp.int32),
        pltpu.VMEM((ITERS, CH), jnp.int32),
        pltpu.VMEM((CH, H), jnp.float32),
        pltpu.VMEM((CH, H), jnp.float32),
        pltpu.VMEM((CH, H), jnp.float32),
        pltpu.VMEM_SHARED((N, H), jnp.float32),
        pltpu.SemaphoreType.DMA,
        pltpu.SemaphoreType.DMA,
        pltpu.SemaphoreType.DMA,
    ],
    compiler_params=pltpu.CompilerParams(use_tc_tiling_on_sc=False),
)


NBLK = 10
BLK = N // NBLK


def _prep_body(degp_ref, x_ref, w_ref, hp_ref, dinv_ref):
    deg = degp_ref[0][:, 0:1] + degp_ref[1][:, 0:1] + 1.0
    dinv = lax.rsqrt(deg)
    dinv_ref[...] = dinv
    hp_ref[...] = (
        jnp.dot(x_ref[...], w_ref[...], preferred_element_type=jnp.float32) * dinv
    )


_prep_call = pl.pallas_call(
    _prep_body,
    grid=(NBLK,),
    in_specs=[
        pl.BlockSpec((2, BLK, 16), lambda i: (0, i, 0)),
        pl.BlockSpec((BLK, F), lambda i: (i, 0)),
        pl.BlockSpec((F, H), lambda i: (0, 0)),
    ],
    out_specs=[
        pl.BlockSpec((BLK, H), lambda i: (i, 0)),
        pl.BlockSpec((BLK, 1), lambda i: (i, 0)),
    ],
    out_shape=[
        jax.ShapeDtypeStruct((N, H), jnp.float32),
        jax.ShapeDtypeStruct((N, 1), jnp.float32),
    ],
)


def _block_t(part_ref, dinv_ref, b_ref):
    return (part_ref[0] + part_ref[1]) * dinv_ref[...] + b_ref[...]


def _bn_y(t, ssum_ref, ssq_ref, g_ref, be_ref):
    mu = ssum_ref[...] * (1.0 / N)
    var = ssq_ref[...] * (1.0 / N) - mu * mu
    return jnp.maximum(
        g_ref[...] * (t - mu) / jnp.sqrt(var + EPS) + be_ref[...], 0.0
    )


def _mid_body(part_ref, dinv_ref, b_ref, g_ref, be_ref, wn_ref, out_ref,
              ssum_ref, ssq_ref):
    i = pl.program_id(0)
    t = _block_t(part_ref, dinv_ref, b_ref)

    @pl.when(i == 0)
    def _():
        ssum_ref[...] = jnp.zeros_like(ssum_ref)
        ssq_ref[...] = jnp.zeros_like(ssq_ref)

    @pl.when(i < NBLK)
    def _():
        ssum_ref[...] += jnp.sum(t, axis=0, keepdims=True)
        ssq_ref[...] += jnp.sum(t * t, axis=0, keepdims=True)

    @pl.when(i >= NBLK)
    def _():
        y = _bn_y(t, ssum_ref, ssq_ref, g_ref, be_ref)
        out_ref[...] = (
            jnp.dot(y, wn_ref[...], preferred_element_type=jnp.float32)
            * dinv_ref[...]
        )


_mid_call = pl.pallas_call(
    _mid_body,
    grid=(2 * NBLK,),
    in_specs=[
        pl.BlockSpec((2, BLK, H), lambda i: (0, i % NBLK, 0)),
        pl.BlockSpec((BLK, 1), lambda i: (i % NBLK, 0)),
        pl.BlockSpec((1, H), lambda i: (0, 0)),
        pl.BlockSpec((1, H), lambda i: (0, 0)),
        pl.BlockSpec((1, H), lambda i: (0, 0)),
        pl.BlockSpec((H, H), lambda i: (0, 0)),
    ],
    out_specs=pl.BlockSpec((BLK, H), lambda i: (i % NBLK, 0)),
    out_shape=jax.ShapeDtypeStruct((N, H), jnp.float32),
    scratch_shapes=[
        pltpu.VMEM((1, H), jnp.float32),
        pltpu.VMEM((1, H), jnp.float32),
    ],
)


def _final_body(part_ref, dinv_ref, b_ref, g_ref, be_ref, batch_ref, wlin_ref,
                blin_ref, out_ref, ssum_ref, ssq_ref, pool_ref, cnt_ref):
    i = pl.program_id(0)
    t = _block_t(part_ref, dinv_ref, b_ref)

    @pl.when(i == 0)
    def _():
        ssum_ref[...] = jnp.zeros_like(ssum_ref)
        ssq_ref[...] = jnp.zeros_like(ssq_ref)
        pool_ref[...] = jnp.zeros_like(pool_ref)
        cnt_ref[...] = jnp.zeros_like(cnt_ref)

    @pl.when(i < NBLK)
    def _():
        ssum_ref[...] += jnp.sum(t, axis=0, keepdims=True)
        ssq_ref[...] += jnp.sum(t * t, axis=0, keepdims=True)

    @pl.when(i >= NBLK)
    def _():
        y = _bn_y(t, ssum_ref, ssq_ref, g_ref, be_ref)
        gid = lax.broadcasted_iota(jnp.int32, (G, BLK), 0)
        ind = (batch_ref[0] == gid).astype(jnp.float32)
        pool_ref[...] += jnp.dot(ind, y, preferred_element_type=jnp.float32)
        cnt_ref[...] += jnp.sum(ind, axis=1, keepdims=True)

    @pl.when(i == 2 * NBLK - 1)
    def _():
        pooled = pool_ref[...] / jnp.maximum(cnt_ref[...], 1.0)
        out_ref[...] = (
            jnp.dot(pooled, wlin_ref[...], preferred_element_type=jnp.float32)
            + blin_ref[...]
        )


_final_call = pl.pallas_call(
    _final_body,
    grid=(2 * NBLK,),
    in_specs=[
        pl.BlockSpec((2, BLK, H), lambda i: (0, i % NBLK, 0)),
        pl.BlockSpec((BLK, 1), lambda i: (i % NBLK, 0)),
        pl.BlockSpec((1, H), lambda i: (0, 0)),
        pl.BlockSpec((1, H), lambda i: (0, 0)),
        pl.BlockSpec((1, H), lambda i: (0, 0)),
        pl.BlockSpec((1, 1, BLK), lambda i: (i % NBLK, 0, 0)),
        pl.BlockSpec((H, 1), lambda i: (0, 0)),
        pl.BlockSpec((1, 1), lambda i: (0, 0)),
    ],
    out_specs=pl.BlockSpec((G, 1), lambda i: (0, 0)),
    out_shape=jax.ShapeDtypeStruct((G, 1), jnp.float32),
    scratch_shapes=[
        pltpu.VMEM((1, H), jnp.float32),
        pltpu.VMEM((1, H), jnp.float32),
        pltpu.VMEM((G, H), jnp.float32),
        pltpu.VMEM((G, 1), jnp.float32),
    ],
)


def kernel(x, edge_index, batch, W0, b0, gamma0, beta0, W1, b1, gamma1, beta1,
           W2, b2, gamma2, beta2, Wlin, blin):
    src = edge_index[0]
    dst = edge_index[1]
    src3 = src.reshape(NC * NS, ITERS, CH)
    dst3 = dst.reshape(NC * NS, ITERS, CH)

    e0 = jnp.zeros((CH, 16), jnp.float32).at[:, 0].set(1.0)
    z16 = jnp.zeros((STG, 16), jnp.float32)
    z128 = jnp.zeros((STG, H), jnp.float32)

    degp = _deg_call(dst3, e0, z16)
    hp, dinv = _prep_call(degp, x, W0)

    b0r, g0r, be0r = b0.reshape(1, H), gamma0.reshape(1, H), beta0.reshape(1, H)
    b1r, g1r, be1r = b1.reshape(1, H), gamma1.reshape(1, H), beta1.reshape(1, H)
    b2r, g2r, be2r = b2.reshape(1, H), gamma2.reshape(1, H), beta2.reshape(1, H)

    part = _scatter_call(hp, src3, dst3, z128)
    hp = _mid_call(part, dinv, b0r, g0r, be0r, W1)
    part = _scatter_call(hp, src3, dst3, z128)
    hp = _mid_call(part, dinv, b1r, g1r, be1r, W2)
    part = _scatter_call(hp, src3, dst3, z128)

    return _final_call(part, dinv, b2r, g2r, be2r, batch.reshape(NBLK, 1, BLK),
                       Wlin, blin.reshape(1, 1))


# speedup vs baseline: 1.0854x; 1.0854x over previous
import functools

import jax
import jax.numpy as jnp
from jax import lax
from jax.experimental import pallas as pl
from jax.experimental.pallas import tpu as pltpu
from jax.experimental.pallas import tpu_sc as plsc

N = 10000
F = 128
H = 128
G = 64
EPS = 1e-5

NC = 2
NS = 16
CH = 80
E_TOTAL = 320000
ITERS = E_TOTAL // (NC * NS) // CH

STG = 80
NSTG = N // STG
STG_ROUNDS = (NSTG + NS - 1) // NS

_MESH = plsc.VectorSubcoreMesh(
    core_axis_name="c", subcore_axis_name="s", num_cores=NC, num_subcores=NS
)


def _deg_body(dst3_hbm, e0_hbm, z16_hbm, degp_hbm, didx_all, ones_v, stage_v,
              dwide):
    c = lax.axis_index("c")
    s = lax.axis_index("s")
    iters = dst3_hbm.shape[1]
    wid = c * NS + s

    pltpu.sync_copy(e0_hbm, ones_v)
    pltpu.sync_copy(z16_hbm, stage_v)
    pltpu.sync_copy(dst3_hbm.at[wid], didx_all)

    if True:
        for ch in range(STG_ROUNDS):
            k = s + NS * ch

            @pl.when(k < NSTG)
            def _():
                pltpu.sync_copy(stage_v, dwide.at[pl.ds(k * STG, STG)])

        plsc.subcore_barrier()

        def body(i, carry):
            pltpu.sync_copy(ones_v, dwide.at[didx_all.at[i]], add=True)
            return carry

        lax.fori_loop(0, iters, body, 0)
        plsc.subcore_barrier()

        for ch in range(STG_ROUNDS):
            k = s + NS * ch

            @pl.when(k < NSTG)
            def _():
                pltpu.sync_copy(dwide.at[pl.ds(k * STG, STG)], stage_v)
                pltpu.sync_copy(stage_v, degp_hbm.at[c, pl.ds(k * STG, STG)])


_deg_call = pl.kernel(
    _deg_body,
    out_type=jax.ShapeDtypeStruct((NC, N, 16), jnp.float32),
    mesh=_MESH,
    scratch_types=[
        pltpu.VMEM((ITERS, CH), jnp.int32),
        pltpu.VMEM((CH, 16), jnp.float32),
        pltpu.VMEM((STG, 16), jnp.float32),
        pltpu.VMEM_SHARED((N, 16), jnp.float32),
    ],
    compiler_params=pltpu.CompilerParams(use_tc_tiling_on_sc=False),
)


def _scatter_body(hp_hbm, src3_hbm, dst3_hbm, z128_hbm, part_hbm,
                  sidx_all, didx_all, rows0, rows1, rows2, acc,
                  semg0, semg1, semg2):
    c = lax.axis_index("c")
    s = lax.axis_index("s")
    iters = src3_hbm.shape[1]
    assert iters % 3 == 2, "edge pipeline expects 3k+2 chunks"
    trips = iters // 3
    wid = c * NS + s

    pltpu.sync_copy(z128_hbm, rows0)
    pltpu.sync_copy(src3_hbm.at[wid], sidx_all)
    pltpu.sync_copy(dst3_hbm.at[wid], didx_all)

    if True:
        for ch in range(STG_ROUNDS):
            k = s + NS * ch

            @pl.when((k < NSTG) & (c == 0))
            def _():
                pltpu.sync_copy(hp_hbm.at[pl.ds(k * STG, STG)], rows0)
                pltpu.sync_copy(rows0, acc.at[pl.ds(k * STG, STG)])

            @pl.when((k < NSTG) & (c != 0))
            def _():
                pltpu.sync_copy(rows0, acc.at[pl.ds(k * STG, STG)])

        plsc.subcore_barrier()

        pltpu.async_copy(hp_hbm.at[sidx_all.at[0]], rows0, semg0)
        pltpu.async_copy(hp_hbm.at[sidx_all.at[1]], rows1, semg1)
        pltpu.async_copy(hp_hbm.at[sidx_all.at[2]], rows2, semg2)

        def step(i, rw, sem, refill):
            pltpu.make_async_copy(hp_hbm.at[sidx_all.at[i]], rw, sem).wait()
            pltpu.sync_copy(rw, acc.at[didx_all.at[i]], add=True)
            if refill is None:
                pltpu.async_copy(hp_hbm.at[sidx_all.at[i + 3]], rw, sem)
            else:
                @pl.when(refill)
                def _():
                    pltpu.async_copy(hp_hbm.at[sidx_all.at[i + 3]], rw, sem)

        def body(j, carry):
            i0 = 3 * j
            step(i0, rows0, semg0, None)
            step(i0 + 1, rows1, semg1, None)
            step(i0 + 2, rows2, semg2, j < trips - 1)
            return carry

        lax.fori_loop(0, trips, body, 0)
        pltpu.make_async_copy(hp_hbm.at[sidx_all.at[iters - 2]], rows0, semg0).wait()
        pltpu.sync_copy(rows0, acc.at[didx_all.at[iters - 2]], add=True)
        pltpu.make_async_copy(hp_hbm.at[sidx_all.at[iters - 1]], rows1, semg1).wait()
        pltpu.sync_copy(rows1, acc.at[didx_all.at[iters - 1]], add=True)
        plsc.subcore_barrier()

        for ch in range(STG_ROUNDS):
            k = s + NS * ch

            @pl.when(k < NSTG)
            def _():
                pltpu.sync_copy(acc.at[pl.ds(k * STG, STG)], rows0)
                pltpu.sync_copy(rows0, part_hbm.at[c, pl.ds(k * STG, STG)])


_scatter_call = pl.kernel(
    _scatter_body,
    out_type=jax.ShapeDtypeStruct((NC, N, H), jnp.float32),
    mesh=_MESH,
    scratch_types=[
        pltpu.VMEM((ITERS, CH), jnp.int32),
        pltpu.VMEM((ITERS, CH), jnp.int32),
        pltpu.VMEM((CH, H), jnp.float32),
        pltpu.VMEM((CH, H), jnp.float32),
        pltpu.VMEM((CH, H), jnp.float32),
        pltpu.VMEM_SHARED((N, H), jnp.float32),
        pltpu.SemaphoreType.DMA,
        pltpu.SemaphoreType.DMA,
        pltpu.SemaphoreType.DMA,
    ],
    compiler_params=pltpu.CompilerParams(use_tc_tiling_on_sc=False),
)


def _prep_body(degp_ref, x_ref, w_ref, hp_ref, dinv_ref):
    deg = degp_ref[0][:, 0:1] + degp_ref[1][:, 0:1] + 1.0
    dinv = lax.rsqrt(deg)
    dinv_ref[...] = dinv
    hp_ref[...] = (
        jnp.dot(x_ref[...], w_ref[...], preferred_element_type=jnp.float32) * dinv
    )


_prep_call = pl.pallas_call(
    _prep_body,
    out_shape=[
        jax.ShapeDtypeStruct((N, H), jnp.float32),
        jax.ShapeDtypeStruct((N, 1), jnp.float32),
    ],
)


def _bn_relu(t, g, be):
    mu = jnp.mean(t, axis=0, keepdims=True)
    var = jnp.mean((t - mu) ** 2, axis=0, keepdims=True)
    return jnp.maximum(g * (t - mu) / jnp.sqrt(var + EPS) + be, 0.0)


def _mid_body(part_ref, dinv_ref, b_ref, g_ref, be_ref, wn_ref, out_ref):
    t = (part_ref[0] + part_ref[1]) * dinv_ref[...] + b_ref[...]
    y = _bn_relu(t, g_ref[...], be_ref[...])
    out_ref[...] = (
        jnp.dot(y, wn_ref[...], preferred_element_type=jnp.float32) * dinv_ref[...]
    )


_mid_call = pl.pallas_call(
    _mid_body,
    out_shape=jax.ShapeDtypeStruct((N, H), jnp.float32),
)


def _final_body(part_ref, dinv_ref, b_ref, g_ref, be_ref, batch_ref, wlin_ref,
                blin_ref, out_ref):
    t = (part_ref[0] + part_ref[1]) * dinv_ref[...] + b_ref[...]
    y = _bn_relu(t, g_ref[...], be_ref[...])
    gid = lax.broadcasted_iota(jnp.int32, (G, N), 0)
    ind = (batch_ref[...] == gid).astype(jnp.float32)
    sums = jnp.dot(ind, y, preferred_element_type=jnp.float32)
    counts = jnp.sum(ind, axis=1, keepdims=True)
    pooled = sums / jnp.maximum(counts, 1.0)
    out_ref[...] = (
        jnp.dot(pooled, wlin_ref[...], preferred_element_type=jnp.float32)
        + blin_ref[...]
    )


_final_call = pl.pallas_call(
    _final_body,
    out_shape=jax.ShapeDtypeStruct((G, 1), jnp.float32),
)


def kernel(x, edge_index, batch, W0, b0, gamma0, beta0, W1, b1, gamma1, beta1,
           W2, b2, gamma2, beta2, Wlin, blin):
    src = edge_index[0]
    dst = edge_index[1]
    src3 = src.reshape(NC * NS, ITERS, CH)
    dst3 = dst.reshape(NC * NS, ITERS, CH)

    e0 = jnp.zeros((CH, 16), jnp.float32).at[:, 0].set(1.0)
    z16 = jnp.zeros((STG, 16), jnp.float32)
    z128 = jnp.zeros((STG, H), jnp.float32)

    degp = _deg_call(dst3, e0, z16)
    hp, dinv = _prep_call(degp, x, W0)

    b0r, g0r, be0r = b0.reshape(1, H), gamma0.reshape(1, H), beta0.reshape(1, H)
    b1r, g1r, be1r = b1.reshape(1, H), gamma1.reshape(1, H), beta1.reshape(1, H)
    b2r, g2r, be2r = b2.reshape(1, H), gamma2.reshape(1, H), beta2.reshape(1, H)

    part = _scatter_call(hp, src3, dst3, z128)
    hp = _mid_call(part, dinv, b0r, g0r, be0r, W1)
    part = _scatter_call(hp, src3, dst3, z128)
    hp = _mid_call(part, dinv, b1r, g1r, be1r, W2)
    part = _scatter_call(hp, src3, dst3, z128)

    return _final_call(part, dinv, b2r, g2r, be2r, batch.reshape(1, N),
                       Wlin, blin.reshape(1, 1))
